# baseline (speedup 1.0000x reference)
import functools

import jax
import jax.numpy as jnp
from jax import lax
from jax.experimental import pallas as pl
from jax.experimental.pallas import tpu as pltpu
from jax.experimental.pallas import tpu_sc as plsc

_B = 16384
_DM = 32
_DS = 16
_NC = 2
_NS = 16
_NW = _NC * _NS
_BPW = _B // _NW
_G = 8
_NGS = _BPW // _G
_NGRP = 7813
_GPW = 245
_CAP = 16
_OVF = 2048
_NB = _B // 16
_R = 4
_DUMP = _B

_mesh = plsc.VectorSubcoreMesh(core_axis_name="c", subcore_axis_name="s")


@functools.partial(
    pl.kernel,
    out_type=(jax.ShapeDtypeStruct((_B + 128, 128), jnp.float32),
              jax.ShapeDtypeStruct((_B, _DS), jnp.float32)),
    mesh=_mesh,
    scratch_types=[
        pltpu.VMEM((_B,), jnp.int32),
        pltpu.VMEM((_BPW + 8,), jnp.int32),
        pltpu.VMEM((_B + 16,), jnp.int32),
        pltpu.VMEM((_B + 16,), jnp.int32),
        pltpu.VMEM((_GPW * _CAP + 16,), jnp.int32),
        pltpu.VMEM((_GPW * _CAP + 16,), jnp.int32),
        pltpu.VMEM((272,), jnp.int32),
        pltpu.VMEM((272,), jnp.int32),
        pltpu.VMEM((_OVF + 16,), jnp.int32),
        pltpu.VMEM((_OVF + 16,), jnp.int32),
        pltpu.VMEM((_R, _DM, 128), jnp.float32),
        pltpu.VMEM((2, 128, 128), jnp.float32),
        pltpu.VMEM((2, 128), jnp.int32),
        pltpu.VMEM((2, _G, 8, _DS), jnp.float32),
        pltpu.VMEM((2, _G, _DS), jnp.float32),
        pltpu.SemaphoreType.DMA,
        pltpu.SemaphoreType.DMA,
        pltpu.SemaphoreType.DMA,
        pltpu.SemaphoreType.DMA,
    ],
    compiler_params=pltpu.CompilerParams(needs_layout_passes=False),
)
def _gather_kernel(mid_hbm, sid_hbm, wmt_hbm, ws_hbm, outm_hbm, outs_hbm,
                   idx_all, idx_s, sel_i, sel_p, bin_i, bin_p, cnts, nelist,
                   ovf_i, ovf_p, ring, stage, stpos, blk_s, rows_s,
                   sem_m, sem_sc, sem_s, sem_o):
    wid = lax.axis_index("s") * _NC + lax.axis_index("c")
    base = wid * _BPW
    lo = wid * _GPW
    iota16 = lax.iota(jnp.int32, 16)
    zeros16 = jnp.zeros((16,), jnp.int32)

    pltpu.sync_copy(mid_hbm, idx_all)
    pltpu.sync_copy(sid_hbm.at[pl.ds(base, _BPW)], idx_s.at[pl.ds(0, _BPW)])

    @pl.loop(0, _NB, init_carry=jnp.int32(0))
    def _scan(v, cnt):
        vec = idx_all[pl.ds(v * 16, 16)]
        c = jax.lax.shift_right_logical(vec, 7) - lo
        m = jax.lax.bitwise_and(c >= 0, c < _GPW)
        plsc.store_compressed(sel_i.at[pl.ds(cnt, 16)], vec, mask=m)
        plsc.store_compressed(sel_p.at[pl.ds(cnt, 16)], iota16 + v * 16, mask=m)
        return cnt + plsc.all_reduce_population_count(m)[0]

    n_sel = _scan

    @pl.loop(0, 17)
    def _zero(v):
        cnts[pl.ds(v * 16, 16)] = zeros16

    def _rmw1(ref, at, val):
        vec = ref[pl.ds(at, 16)]
        ref[pl.ds(at, 16)] = jnp.where(iota16 == 0, val, vec)

    @pl.loop(0, n_sel, init_carry=jnp.int32(0))
    def _bin(j, ocnt):
        idx = sel_i[pl.ds(j, 16)][0]
        pos = sel_p[pl.ds(j, 16)][0]
        c = jax.lax.shift_right_logical(idx, 7) - lo
        n = cnts[pl.ds(c, 16)][0]

        @pl.when(n < _CAP)
        def _():
            _rmw1(bin_i, c * _CAP + n, idx)
            _rmw1(bin_p, c * _CAP + n, pos)

        @pl.when(jnp.logical_and(n >= _CAP, ocnt < _OVF))
        def _():
            _rmw1(ovf_i, ocnt, idx)
            _rmw1(ovf_p, ocnt, pos)

        _rmw1(cnts, c, n + 1)
        grew = jnp.where(n >= _CAP, jnp.where(ocnt < _OVF, 1, 0), 0)
        return ocnt + grew.astype(jnp.int32)

    n_ovf = _bin

    @pl.loop(0, 16, init_carry=jnp.int32(0))
    def _ne(v, cnt):
        gids = iota16 + v * 16
        cv = cnts[pl.ds(v * 16, 16)]
        m = jax.lax.bitwise_and(cv > 0, gids < _GPW)
        plsc.store_compressed(nelist.at[pl.ds(cnt, 16)], gids, mask=m)
        return cnt + plsc.all_reduce_population_count(m)[0]

    n_ne = _ne

    def _blk_off(g):
        return jnp.minimum((lo + g) * 128, 1000000 - 128)

    def fire_mat(i):
        g = nelist[pl.ds(i, 16)][0]
        off = pl.multiple_of(_blk_off(g), 128)
        pltpu.async_copy(wmt_hbm.at[:, pl.ds(off, 128)],
                         ring.at[jax.lax.rem(i, _R)], sem_m)

    def stash_row2(idx, pos, off_col, scnt, blk):
        lane = jnp.full((16,), idx - off_col, jnp.int32)
        sslot = jax.lax.bitwise_and(jax.lax.shift_right_logical(scnt, 7), 1)
        row = jax.lax.bitwise_and(scnt, 127)

        @pl.when(jnp.logical_and(row == 0, scnt >= 256))
        def _():
            pltpu.make_async_copy(
                stage.at[0], outm_hbm.at[pl.ds(_DUMP, 128)], sem_sc).wait()

        stage[sslot, row, pl.ds(0, 16)] = plsc.load_gather(blk, [iota16, lane])
        stage[sslot, row, pl.ds(16, 16)] = plsc.load_gather(
            blk, [iota16 + 16, lane])
        _rmw1(stpos.at[sslot], row, pos)
        return row == 127

    def fire_spc(g, slot):
        ts_vec = jax.lax.shift_right_logical(idx_s[pl.ds(g * _G, 16)], 3)
        for k in range(_G):
            pltpu.async_copy(ws_hbm.at[ts_vec[k]], blk_s.at[slot, k], sem_s)

    def drain_extract_spc(g, slot):
        rs_vec = jax.lax.bitwise_and(idx_s[pl.ds(g * _G, 16)], 7)
        pltpu.make_async_copy(ws_hbm.at[pl.ds(0, _G)], blk_s.at[slot],
                              sem_s).wait()
        for k in range(_G):
            rows_s[slot, k, pl.ds(0, _DS)] = blk_s[slot, k, rs_vec[k],
                                                   pl.ds(0, _DS)]
        pltpu.async_copy(rows_s.at[slot],
                         outs_hbm.at[pl.ds(base + g * _G, _G)], sem_o)

    for i in range(_R - 1):
        @pl.when(i < n_ne)
        def _():
            fire_mat(i)
    fire_spc(0, 0)

    n_iter = jnp.maximum(n_ne, _NGS)

    @pl.loop(0, n_iter, init_carry=(jnp.int32(0), jnp.int32(0)))
    def _main(i, carry):
        scnt, nfired = carry

        @pl.when(i + _R - 1 < n_ne)
        def _():
            fire_mat(i + _R - 1)

        @pl.when(i + 1 < _NGS)
        def _():
            fire_spc(i + 1, jax.lax.bitwise_and(i + 1, 1))

        @pl.when(i < n_ne)
        def _():
            pltpu.make_async_copy(
                wmt_hbm.at[:, pl.ds(0, 128)],
                ring.at[jax.lax.rem(i, _R)], sem_m).wait()

        @pl.when(jnp.logical_and(i >= 2, i < _NGS))
        def _():
            pltpu.make_async_copy(
                rows_s.at[0], outs_hbm.at[pl.ds(base, _G)], sem_o).wait()

        @pl.when(i < _NGS)
        def _():
            drain_extract_spc(i, jax.lax.bitwise_and(i, 1))

        g = jax.lax.bitwise_and(
            nelist[pl.ds(jnp.minimum(i, n_ne), 16)][0], 255)
        n_eff = jnp.where(
            i < n_ne, jnp.minimum(cnts[pl.ds(g, 16)][0], _CAP), 0)
        off_col = _blk_off(g)
        blk = ring.at[jax.lax.rem(i, _R)]

        @pl.loop(0, n_eff, init_carry=(scnt, nfired))
        def _inner(j, c2):
            s2, f2 = c2
            idx = bin_i[pl.ds(g * _CAP + j, 16)][0]
            pos = bin_p[pl.ds(g * _CAP + j, 16)][0]
            full = stash_row2(idx, pos, off_col, s2, blk)
            f3 = jnp.where(full, f2 + 1, f2)

            @pl.when(full)
            def _():
                sslot = jax.lax.bitwise_and(
                    jax.lax.shift_right_logical(s2, 7), 1)
                pltpu.async_copy(stage.at[sslot],
                                 outm_hbm.at[stpos.at[sslot]], sem_sc)

            return s2 + 1, f3

        return _inner

    scnt_f, nfired_f = _main

    @pl.loop(0, n_ovf, init_carry=(scnt_f, nfired_f))
    def _ovfl(k, carry):
        scnt0, nfired0 = carry
        idx = ovf_i[pl.ds(k, 16)][0]
        pos = ovf_p[pl.ds(k, 16)][0]
        goff = jnp.minimum(jax.lax.shift_right_logical(idx, 7) * 128,
                           1000000 - 128)
        off = pl.multiple_of(goff, 128)
        pltpu.async_copy(wmt_hbm.at[:, pl.ds(off, 128)], ring.at[0], sem_m)
        pltpu.make_async_copy(wmt_hbm.at[:, pl.ds(0, 128)], ring.at[0],
                              sem_m).wait()
        full = stash_row2(idx, pos, goff, scnt0, ring.at[0])
        f3 = jnp.where(full, nfired0 + 1, nfired0)

        @pl.when(full)
        def _():
            sslot = jax.lax.bitwise_and(
                jax.lax.shift_right_logical(scnt0, 7), 1)
            pltpu.async_copy(stage.at[sslot],
                             outm_hbm.at[stpos.at[sslot]], sem_sc)

        return scnt0 + 1, f3

    scnt_f2, nfired_f2 = _ovfl

    row_f = jax.lax.bitwise_and(scnt_f2, 127)

    @pl.when(row_f > 0)
    def _():
        sslot = jax.lax.bitwise_and(jax.lax.shift_right_logical(scnt_f2, 7), 1)

        @pl.loop(0, 8)
        def _pad(v):
            vec = stpos[sslot, pl.ds(v * 16, 16)]
            mask = (iota16 + v * 16) >= row_f
            stpos[sslot, pl.ds(v * 16, 16)] = jnp.where(
                mask, jnp.full((16,), _DUMP + wid, jnp.int32), vec)

        pltpu.async_copy(stage.at[sslot], outm_hbm.at[stpos.at[sslot]], sem_sc)

    nfired_f3 = jnp.where(row_f > 0, nfired_f2 + 1, nfired_f2)

    @pl.loop(0, jnp.minimum(nfired_f3, 2))
    def _dr(k):
        pltpu.make_async_copy(
            stage.at[0], outm_hbm.at[pl.ds(_DUMP, 128)], sem_sc).wait()

    pltpu.make_async_copy(rows_s.at[0], outs_hbm.at[pl.ds(base, _G)],
                          sem_o).wait()
    pltpu.make_async_copy(rows_s.at[1], outs_hbm.at[pl.ds(base, _G)],
                          sem_o).wait()


def kernel(material_id, space_group, W_material, W_space):
    wmt = W_material.T
    ws3 = W_space.reshape(12500, 8, _DS)
    outm, outs = _gather_kernel(material_id.astype(jnp.int32),
                                space_group.astype(jnp.int32), wmt, ws3)
    return jnp.concatenate([outm[:_B, :_DM], outs], axis=-1)


# speedup vs baseline: 1.0343x; 1.0343x over previous
import functools

import jax
import jax.numpy as jnp
from jax import lax
from jax.experimental import pallas as pl
from jax.experimental.pallas import tpu as pltpu
from jax.experimental.pallas import tpu_sc as plsc

_B = 16384
_DM = 32
_DS = 16
_D = _DM + _DS
_NC = 2
_NS = 16
_NW = _NC * _NS
_BPW = _B // _NW
_G = 8
_NG = _BPW // _G

_mesh = plsc.VectorSubcoreMesh(core_axis_name="c", subcore_axis_name="s")


@functools.partial(
    pl.kernel,
    out_type=jax.ShapeDtypeStruct((_B, _D), jnp.float32),
    mesh=_mesh,
    scratch_types=[
        pltpu.VMEM((_BPW + 8,), jnp.int32),
        pltpu.VMEM((_BPW + 8,), jnp.int32),
        pltpu.VMEM((2, _G, _DM, 128), jnp.float32),
        pltpu.VMEM((2, _G, 8, _DS), jnp.float32),
        pltpu.VMEM((2, _G, _D), jnp.float32),
        pltpu.SemaphoreType.DMA((2,)),
        pltpu.SemaphoreType.DMA((2,)),
        pltpu.SemaphoreType.DMA((2,)),
    ],
    compiler_params=pltpu.CompilerParams(needs_layout_passes=False),
)
def _gather_kernel(mid_hbm, sid_hbm, wmt_hbm, ws_hbm, out_hbm,
                   idx_m, idx_s, blk_m, blk_s, rows_v,
                   sem_m, sem_s, sem_o):
    wid = lax.axis_index("s") * _NC + lax.axis_index("c")
    base = wid * _BPW
    pltpu.sync_copy(mid_hbm.at[pl.ds(base, _BPW)], idx_m.at[pl.ds(0, _BPW)])
    pltpu.sync_copy(sid_hbm.at[pl.ds(base, _BPW)], idx_s.at[pl.ds(0, _BPW)])

    def fire(g, slot):
        cm_vec = jax.lax.shift_right_logical(idx_m[pl.ds(g * _G, 16)], 7)
        ts_vec = jax.lax.shift_right_logical(idx_s[pl.ds(g * _G, 16)], 3)
        for k in range(_G):
            off = pl.multiple_of(cm_vec[k] * 128, 128)
            pltpu.async_copy(
                wmt_hbm.at[:, pl.ds(off, 128)], blk_m.at[slot, k],
                sem_m.at[slot])
            pltpu.async_copy(ws_hbm.at[ts_vec[k]], blk_s.at[slot, k],
                             sem_s.at[slot])

    iota16 = lax.iota(jnp.int32, 16)

    def drain_extract(g, slot):
        lm_vec = jax.lax.bitwise_and(idx_m[pl.ds(g * _G, 16)], 127)
        rs_vec = jax.lax.bitwise_and(idx_s[pl.ds(g * _G, 16)], 7)
        for k in range(_G):
            pltpu.make_async_copy(
                wmt_hbm.at[:, pl.ds(0, 128)], blk_m.at[slot, k],
                sem_m.at[slot]).wait()
        pltpu.make_async_copy(ws_hbm.at[pl.ds(0, _G)], blk_s.at[slot],
                              sem_s.at[slot]).wait()
        for k in range(_G):
            lane = jnp.full((16,), lm_vec[k], jnp.int32)
            rows_v[slot, k, pl.ds(0, 16)] = plsc.load_gather(
                blk_m.at[slot, k], [iota16, lane])
            rows_v[slot, k, pl.ds(16, 16)] = plsc.load_gather(
                blk_m.at[slot, k], [iota16 + 16, lane])
            rows_v[slot, k, pl.ds(_DM, 16)] = blk_s[slot, k, rs_vec[k], pl.ds(0, 16)]
        pltpu.async_copy(rows_v.at[slot],
                         out_hbm.at[pl.ds(base + g * _G, _G)], sem_o.at[slot])

    fire(0, 0)

    @pl.loop(0, _NG)
    def _body(c):
        slot = jax.lax.bitwise_and(c, 1)
        nslot = jax.lax.bitwise_and(c + 1, 1)

        @pl.when(c + 1 < _NG)
        def _():
            fire(c + 1, nslot)

        @pl.when(c >= 2)
        def _():
            pltpu.make_async_copy(
                rows_v.at[slot], out_hbm.at[pl.ds(base, _G)],
                sem_o.at[slot]).wait()

        drain_extract(c, slot)

    pltpu.make_async_copy(rows_v.at[0], out_hbm.at[pl.ds(base, _G)],
                          sem_o.at[0]).wait()
    pltpu.make_async_copy(rows_v.at[1], out_hbm.at[pl.ds(base, _G)],
                          sem_o.at[1]).wait()


def kernel(material_id, space_group, W_material, W_space):
    wmt = W_material.T
    ws3 = W_space.reshape(12500, 8, _DS)
    return _gather_kernel(material_id.astype(jnp.int32),
                          space_group.astype(jnp.int32), wmt, ws3)


# speedup vs baseline: 1.1772x; 1.1381x over previous
import functools

import jax
import jax.numpy as jnp
from jax import lax
from jax.experimental import pallas as pl
from jax.experimental.pallas import tpu as pltpu
from jax.experimental.pallas import tpu_sc as plsc

_B = 16384
_DM = 32
_DS = 16
_D = _DM + _DS
_NC = 2
_NS = 16
_NW = _NC * _NS
_BPW = _B // _NW
_G = 8
_NG = _BPW // _G

_mesh = plsc.VectorSubcoreMesh(core_axis_name="c", subcore_axis_name="s")


@functools.partial(
    pl.kernel,
    out_type=jax.ShapeDtypeStruct((_B, _D), jnp.float32),
    mesh=_mesh,
    scratch_types=[
        pltpu.VMEM((_BPW + 8,), jnp.int32),
        pltpu.VMEM((_BPW + 8,), jnp.int32),
        pltpu.VMEM((2, _G, _DM, 128), jnp.float32),
        pltpu.VMEM((2, _G, 8, _DS), jnp.float32),
        pltpu.VMEM((2, _G, _D), jnp.float32),
        pltpu.SemaphoreType.DMA,
        pltpu.SemaphoreType.DMA,
        pltpu.SemaphoreType.DMA,
    ],
    compiler_params=pltpu.CompilerParams(needs_layout_passes=False),
)
def _gather_kernel(mid_hbm, sid_hbm, wmt_hbm, ws_hbm, out_hbm,
                   idx_m, idx_s, blk_m, blk_s, rows_v,
                   sem_m, sem_s, sem_o):
    wid = lax.axis_index("s") * _NC + lax.axis_index("c")
    base = wid * _BPW
    pltpu.sync_copy(mid_hbm.at[pl.ds(base, _BPW)], idx_m.at[pl.ds(0, _BPW)])
    pltpu.sync_copy(sid_hbm.at[pl.ds(base, _BPW)], idx_s.at[pl.ds(0, _BPW)])

    def fire(g, slot):
        cm_vec = jax.lax.shift_right_logical(idx_m[pl.ds(g * _G, 16)], 7)
        ts_vec = jax.lax.shift_right_logical(idx_s[pl.ds(g * _G, 16)], 3)
        for k in range(_G):
            off = pl.multiple_of(cm_vec[k] * 128, 128)
            pltpu.async_copy(
                wmt_hbm.at[:, pl.ds(off, 128)], blk_m.at[slot, k], sem_m)
            pltpu.async_copy(ws_hbm.at[ts_vec[k]], blk_s.at[slot, k], sem_s)

    iota16 = lax.iota(jnp.int32, 16)

    def drain_extract(g, slot):
        lm_vec = jax.lax.bitwise_and(idx_m[pl.ds(g * _G, 16)], 127)
        rs_vec = jax.lax.bitwise_and(idx_s[pl.ds(g * _G, 16)], 7)
        for k in range(_G):
            pltpu.make_async_copy(
                wmt_hbm.at[:, pl.ds(0, 128)], blk_m.at[slot, k], sem_m).wait()
        pltpu.make_async_copy(ws_hbm.at[pl.ds(0, _G)], blk_s.at[slot], sem_s).wait()
        for k in range(_G):
            lane = jnp.full((16,), lm_vec[k], jnp.int32)
            rows_v[slot, k, pl.ds(0, 16)] = plsc.load_gather(
                blk_m.at[slot, k], [iota16, lane])
            rows_v[slot, k, pl.ds(16, 16)] = plsc.load_gather(
                blk_m.at[slot, k], [iota16 + 16, lane])
            rows_v[slot, k, pl.ds(_DM, 16)] = blk_s[slot, k, rs_vec[k], pl.ds(0, 16)]
        pltpu.async_copy(rows_v.at[slot], out_hbm.at[pl.ds(base + g * _G, _G)], sem_o)

    fire(0, 0)

    @pl.loop(0, _NG)
    def _body(c):
        slot = jax.lax.bitwise_and(c, 1)
        nslot = jax.lax.bitwise_and(c + 1, 1)

        @pl.when(c + 1 < _NG)
        def _():
            fire(c + 1, nslot)

        @pl.when(c >= 2)
        def _():
            pltpu.make_async_copy(
                rows_v.at[slot], out_hbm.at[pl.ds(base, _G)], sem_o).wait()

        drain_extract(c, slot)

    pltpu.make_async_copy(rows_v.at[0], out_hbm.at[pl.ds(base, _G)], sem_o).wait()
    pltpu.make_async_copy(rows_v.at[1], out_hbm.at[pl.ds(base, _G)], sem_o).wait()


def kernel(material_id, space_group, W_material, W_space):
    wmt = W_material.T
    ws3 = W_space.reshape(12500, 8, _DS)
    return _gather_kernel(material_id.astype(jnp.int32),
                          space_group.astype(jnp.int32), wmt, ws3)
